# baseline (speedup 1.0000x reference)
import functools
import math

import jax
import jax.numpy as jnp
from jax import lax
from jax.experimental import pallas as pl
from jax.experimental.pallas import tpu as pltpu
from jax.experimental.pallas import tpu_sc as plsc

_D = 64
_SCALE = math.sqrt(_D)


@functools.partial(jax.jit, static_argnames=())
def _embed(idx, table):
    (B,) = idx.shape
    info = plsc.get_sparse_core_info()
    nw = info.num_cores * info.num_subcores
    b_per_w = B // nw
    chunk = 512
    n_chunks = b_per_w // chunk

    mesh = plsc.VectorSubcoreMesh(core_axis_name="c", subcore_axis_name="s")

    @functools.partial(
        pl.kernel,
        out_type=jax.ShapeDtypeStruct((B, _D), jnp.float32),
        mesh=mesh,
        scratch_types=[
            pltpu.VMEM((chunk,), jnp.int32),
            pltpu.VMEM((chunk, _D), jnp.float32),
            pltpu.SemaphoreType.DMA,
        ],
        compiler_params=pltpu.CompilerParams(use_tc_tiling_on_sc=False),
    )
    def emb(idx_hbm, table_hbm, out_hbm, idx_v, rows_v, sem):
        wid = lax.axis_index("s") * info.num_cores + lax.axis_index("c")
        base = wid * b_per_w

        def chunk_body(ci, carry):
            off = base + ci * chunk
            pltpu.sync_copy(idx_hbm.at[pl.ds(off, chunk)], idx_v)
            pltpu.async_copy(table_hbm.at[idx_v], rows_v, sem).wait()

            def scale_row(r, c2):
                for j in range(_D // 16):
                    sl = pl.ds(j * 16, 16)
                    rows_v[r, sl] = rows_v[r, sl] * _SCALE
                return c2

            lax.fori_loop(0, chunk, scale_row, 0, unroll=4)
            pltpu.sync_copy(rows_v, out_hbm.at[pl.ds(off, chunk)])
            return carry

        lax.fori_loop(0, n_chunks, chunk_body, 0)

    return emb(idx, table)


def kernel(input, table):
    b, s = input.shape
    idx = input.reshape(b * s).astype(jnp.int32)
    out = _embed(idx, table)
    return out.reshape(b, s, _D)


# speedup vs baseline: 1.0890x; 1.0890x over previous
import functools
import math

import jax
import jax.numpy as jnp
from jax import lax
from jax.experimental import pallas as pl
from jax.experimental.pallas import tpu as pltpu
from jax.experimental.pallas import tpu_sc as plsc

_D = 64
_SCALE = math.sqrt(_D)


def _embed(idx, table):
    (B,) = idx.shape
    info = plsc.get_sparse_core_info()
    nw = info.num_cores * info.num_subcores
    b_per_w = B // nw
    chunk = 512
    n_chunks = b_per_w // chunk

    mesh = plsc.VectorSubcoreMesh(core_axis_name="c", subcore_axis_name="s")

    @functools.partial(
        pl.kernel,
        out_type=jax.ShapeDtypeStruct((B, _D), jnp.float32),
        mesh=mesh,
        scratch_types=[
            pltpu.VMEM((b_per_w,), jnp.int32),
            pltpu.VMEM((chunk, _D), jnp.float32),
            pltpu.VMEM((chunk, _D), jnp.float32),
            pltpu.SemaphoreType.DMA,
            pltpu.SemaphoreType.DMA,
            pltpu.SemaphoreType.DMA,
            pltpu.SemaphoreType.DMA,
        ],
        compiler_params=pltpu.CompilerParams(use_tc_tiling_on_sc=False),
    )
    def emb(idx_hbm, table_hbm, out_hbm, idx_v, rows0, rows1,
            gsem0, gsem1, ssem0, ssem1):
        wid = lax.axis_index("s") * info.num_cores + lax.axis_index("c")
        base = wid * b_per_w
        bufs = ((rows0, gsem0, ssem0), (rows1, gsem1, ssem1))

        def gather(c, rows, sem):
            return pltpu.make_async_copy(
                table_hbm.at[idx_v.at[pl.ds(c * chunk, chunk)]], rows, sem)

        def store(c, rows, sem):
            return pltpu.make_async_copy(
                rows, out_hbm.at[pl.ds(base + c * chunk, chunk)], sem)

        pltpu.sync_copy(idx_hbm.at[pl.ds(base, b_per_w)], idx_v)
        gather(0, rows0, gsem0).start()

        @pl.loop(0, n_chunks, step=2)
        def _(ci):
            for b in range(2):
                cur = ci + b
                rows, gsem, ssem = bufs[b]
                nrows, ngsem, nssem = bufs[1 - b]
                nxt = cur + 1

                @pl.when(nxt < n_chunks)
                def _():
                    @pl.when(nxt >= 2)
                    def _():
                        store(nxt - 2, nrows, nssem).wait()

                    gather(nxt, nrows, ngsem).start()

                gather(cur, rows, gsem).wait()

                def scale_row(r, c2):
                    for j in range(_D // 16):
                        sl = pl.ds(j * 16, 16)
                        rows[r, sl] = rows[r, sl] * _SCALE
                    return c2

                lax.fori_loop(0, chunk, scale_row, 0, unroll=4)
                store(cur, rows, ssem).start()

        store(n_chunks - 2, rows0, ssem0).wait()
        store(n_chunks - 1, rows1, ssem1).wait()

    return emb(idx, table)


def kernel(input, table):
    b, s = input.shape
    idx = input.reshape(b * s).astype(jnp.int32)
    out = _embed(idx, table)
    return out.reshape(b, s, _D)
